# Initial kernel scaffold; baseline (speedup 1.0000x reference)
import jax
import jax.numpy as jnp
from jax.experimental import pallas as pl


def kernel(adj, weight_vector, id_embedding):
    raise NotImplementedError("write your pallas kernel here")


# speedup vs baseline: 34.4003x; 34.4003x over previous
import functools

import jax
import jax.numpy as jnp
from jax import lax
from jax.experimental import pallas as pl
from jax.experimental.pallas import tpu as pltpu
from jax.experimental.pallas import tpu_sc as plsc

N_NODES = 50000
N_PAD = 51200
N_EDGES = 800000
E_PAD = 819200
EDGES_PER_TILE = E_PAD // 32
CHUNK = 3200
NCHUNK = EDGES_PER_TILE // CHUNK
ROW_BLOCK = 1024
L = 16


def _s0_body(x_ref, o_ref):
    x = x_ref[...]
    rs = jnp.sum(x, axis=1)
    nrm = jnp.sqrt(jnp.sum(x * x, axis=1))
    o_ref[...] = rs / jnp.maximum(nrm, 1e-12)


def _compute_s0(emb_pad):
    return pl.pallas_call(
        _s0_body,
        grid=(N_PAD // ROW_BLOCK,),
        in_specs=[pl.BlockSpec((ROW_BLOCK, emb_pad.shape[1]), lambda i: (i, 0))],
        out_specs=pl.BlockSpec((ROW_BLOCK,), lambda i: (i,)),
        out_shape=jax.ShapeDtypeStruct((N_PAD,), jnp.float32),
    )(emb_pad)


def _edge_body(src_hbm, dst_hbm, w_hbm, s0_hbm, s1p_hbm, ap_hbm,
               s0_v, src_v, dst_v, w_v, msg_v, s1_sh, a_sh):
    core = lax.axis_index("c")
    sid = lax.axis_index("s")
    wid = sid * 2 + core
    my_slice = pl.ds(sid * (N_PAD // 16), N_PAD // 16)

    def zbody(i, _):
        msg_v[pl.ds(i * L, L)] = jnp.zeros((L,), jnp.float32)
        return 0
    lax.fori_loop(0, CHUNK // L, zbody, 0)
    pltpu.sync_copy(msg_v.at[pl.ds(0, N_PAD // 16)], s1_sh.at[my_slice])
    pltpu.sync_copy(msg_v.at[pl.ds(0, N_PAD // 16)], a_sh.at[my_slice])

    pltpu.sync_copy(s0_hbm, s0_v)
    plsc.subcore_barrier()

    base = wid * EDGES_PER_TILE
    for c in range(NCHUNK):
        off = base + c * CHUNK
        pltpu.sync_copy(src_hbm.at[pl.ds(off, CHUNK)], src_v)
        pltpu.sync_copy(dst_hbm.at[pl.ds(off, CHUNK)], dst_v)
        pltpu.sync_copy(w_hbm.at[pl.ds(off, CHUNK)], w_v)

        def gbody(i, _):
            o = i * L
            sv = src_v[pl.ds(o, L)]
            vals = plsc.load_gather(s0_v, [sv])
            msg_v[pl.ds(o, L)] = w_v[pl.ds(o, L)] * vals
            return 0
        lax.fori_loop(0, CHUNK // L, gbody, 0)

        pltpu.sync_copy(msg_v, s1_sh.at[dst_v], add=True)
        pltpu.sync_copy(w_v, a_sh.at[src_v], add=True)

    plsc.subcore_barrier()
    pltpu.sync_copy(s1_sh.at[my_slice], s1p_hbm.at[core, my_slice])
    pltpu.sync_copy(a_sh.at[my_slice], ap_hbm.at[core, my_slice])


def _edge_pass(src_pad, dst_pad, w_pad, s0):
    mesh = plsc.VectorSubcoreMesh(core_axis_name="c", subcore_axis_name="s")
    f = pl.kernel(
        _edge_body,
        out_type=(
            jax.ShapeDtypeStruct((2, N_PAD), jnp.float32),
            jax.ShapeDtypeStruct((2, N_PAD), jnp.float32),
        ),
        mesh=mesh,
        compiler_params=pltpu.CompilerParams(needs_layout_passes=False),
        scratch_types=[
            pltpu.VMEM((N_PAD,), jnp.float32),
            pltpu.VMEM((CHUNK,), jnp.int32),
            pltpu.VMEM((CHUNK,), jnp.int32),
            pltpu.VMEM((CHUNK,), jnp.float32),
            pltpu.VMEM((CHUNK,), jnp.float32),
            pltpu.VMEM_SHARED((N_PAD,), jnp.float32),
            pltpu.VMEM_SHARED((N_PAD,), jnp.float32),
        ],
    )
    return f(src_pad, dst_pad, w_pad, s0)


def _final_body(s0_ref, s1_ref, a_ref, o_ref):
    s0 = s0_ref[...]
    s1 = s1_ref[0, :] + s1_ref[1, :]
    a = a_ref[0, :] + a_ref[1, :]
    tot = jnp.sum(s0) + jnp.sum(s1) + jnp.sum(s1 * a)
    o_ref[...] = jnp.reshape(tot, (1, 1))


def _final_reduce(s0, s1p, ap):
    return pl.pallas_call(
        _final_body,
        out_shape=jax.ShapeDtypeStruct((1, 1), jnp.float32),
    )(s0, s1p, ap)


def kernel(adj, weight_vector, id_embedding):
    src = adj[0]
    dst = adj[1]
    w = weight_vector[:, 0]

    epad = E_PAD - N_EDGES
    src_pad = jnp.concatenate([src, jnp.full((epad,), N_NODES, jnp.int32)])
    dst_pad = jnp.concatenate([dst, jnp.full((epad,), N_NODES, jnp.int32)])
    w_pad = jnp.concatenate([w, jnp.zeros((epad,), jnp.float32)])
    emb_pad = jnp.pad(id_embedding, ((0, N_PAD - N_NODES), (0, 0)))

    s0 = _compute_s0(emb_pad)
    s1p, ap = _edge_pass(src_pad, dst_pad, w_pad, s0)
    out = _final_reduce(s0, s1p, ap)
    return out[0, 0]


# speedup vs baseline: 47.4385x; 1.3790x over previous
import jax
import jax.numpy as jnp
from jax import lax
from jax.experimental import pallas as pl
from jax.experimental.pallas import tpu as pltpu
from jax.experimental.pallas import tpu_sc as plsc

N_NODES = 50000
N_EDGES = 800000
NBLK = N_EDGES // 128
BLK_LO = NBLK // 32
BLK_HI = BLK_LO + 1
CHUNK = 3200
NFULL = 7
TAIL_HI = BLK_HI * 128 - NFULL * CHUNK
TAIL_LO = BLK_LO * 128 - NFULL * CHUNK
N_ACC = 50048
ZSLICE = N_ACC // 16
ROW_BLOCK = 1024
L = 16


def _s0_body(x_ref, o_ref):
    x = x_ref[...]
    rs = jnp.sum(x, axis=1)
    nrm = jnp.sqrt(jnp.sum(x * x, axis=1))
    o_ref[...] = rs / jnp.maximum(nrm, 1e-12)


def _compute_s0(emb):
    grid = (N_NODES + ROW_BLOCK - 1) // ROW_BLOCK
    return pl.pallas_call(
        _s0_body,
        grid=(grid,),
        in_specs=[pl.BlockSpec((ROW_BLOCK, emb.shape[1]), lambda i: (i, 0))],
        out_specs=pl.BlockSpec((ROW_BLOCK,), lambda i: (i,)),
        out_shape=jax.ShapeDtypeStruct((N_NODES,), jnp.float32),
    )(emb)


def _edge_body(adj_hbm, w_hbm, s0_hbm, s1p0_hbm, s1p1_hbm, ap0_hbm, ap1_hbm,
               s0_v, ed_v, src_v, dst_v, w_v, msg_v, s1_sh, a_sh):
    core = lax.axis_index("c")
    sid = lax.axis_index("s")
    wid = sid * 2 + core

    def zbody(i, _):
        msg_v[pl.ds(i * L, L)] = jnp.zeros((L,), jnp.float32)
        return 0
    lax.fori_loop(0, CHUNK // L, zbody, 0)

    zs = pl.ds(sid * ZSLICE, ZSLICE)
    pltpu.sync_copy(msg_v.at[pl.ds(0, ZSLICE)], s1_sh.at[zs])
    pltpu.sync_copy(msg_v.at[pl.ds(0, ZSLICE)], a_sh.at[zs])

    pltpu.sync_copy(s0_hbm, s0_v)
    plsc.subcore_barrier()

    base = (wid * BLK_LO + jnp.minimum(wid, 10)) * 128

    def do_chunk(off, nedge):
        pltpu.sync_copy(adj_hbm.at[:, pl.ds(off, nedge)], ed_v.at[:, pl.ds(0, nedge)])
        pltpu.sync_copy(w_hbm.at[pl.ds(off, nedge)], w_v.at[pl.ds(0, nedge)])
        def gbody(i, _):
            o = i * L
            sv = ed_v[0, pl.ds(o, L)]
            dv = ed_v[1, pl.ds(o, L)]
            src_v[pl.ds(o, L)] = sv
            dst_v[pl.ds(o, L)] = dv
            vals = plsc.load_gather(s0_v, [sv])
            msg_v[pl.ds(o, L)] = w_v[pl.ds(o, L)] * vals
            return 0
        lax.fori_loop(0, nedge // L, gbody, 0)

        pltpu.sync_copy(msg_v.at[pl.ds(0, nedge)], s1_sh.at[dst_v.at[pl.ds(0, nedge)]], add=True)
        pltpu.sync_copy(w_v.at[pl.ds(0, nedge)], a_sh.at[src_v.at[pl.ds(0, nedge)]], add=True)

    for c in range(NFULL):
        do_chunk(base + c * CHUNK, CHUNK)

    tail_off = base + NFULL * CHUNK

    @pl.when(wid < 10)
    def _tail_hi():
        do_chunk(tail_off, TAIL_HI)

    @pl.when(wid >= 10)
    def _tail_lo():
        do_chunk(tail_off, TAIL_LO)

    plsc.subcore_barrier()

    pltpu.sync_copy(s1_sh.at[zs], msg_v.at[pl.ds(0, ZSLICE)])
    pltpu.sync_copy(a_sh.at[zs], w_v.at[pl.ds(0, ZSLICE)])

    @pl.when(core == 0)
    def _pub0():
        pltpu.sync_copy(msg_v.at[pl.ds(0, ZSLICE)], s1p0_hbm.at[zs])
        pltpu.sync_copy(w_v.at[pl.ds(0, ZSLICE)], ap0_hbm.at[zs])

    @pl.when(core == 1)
    def _pub1():
        pltpu.sync_copy(msg_v.at[pl.ds(0, ZSLICE)], s1p1_hbm.at[zs])
        pltpu.sync_copy(w_v.at[pl.ds(0, ZSLICE)], ap1_hbm.at[zs])


def _edge_pass(adj, w, s0):
    mesh = plsc.VectorSubcoreMesh(core_axis_name="c", subcore_axis_name="s")
    f = pl.kernel(
        _edge_body,
        out_type=(
            jax.ShapeDtypeStruct((N_ACC,), jnp.float32),
            jax.ShapeDtypeStruct((N_ACC,), jnp.float32),
            jax.ShapeDtypeStruct((N_ACC,), jnp.float32),
            jax.ShapeDtypeStruct((N_ACC,), jnp.float32),
        ),
        mesh=mesh,
        compiler_params=pltpu.CompilerParams(needs_layout_passes=False),
        scratch_types=[
            pltpu.VMEM((N_NODES,), jnp.float32),
            pltpu.VMEM((2, CHUNK), jnp.int32),
            pltpu.VMEM((CHUNK,), jnp.int32),
            pltpu.VMEM((CHUNK,), jnp.int32),
            pltpu.VMEM((CHUNK,), jnp.float32),
            pltpu.VMEM((CHUNK,), jnp.float32),
            pltpu.VMEM_SHARED((N_ACC,), jnp.float32),
            pltpu.VMEM_SHARED((N_ACC,), jnp.float32),
        ],
    )
    return f(adj, w, s0)


def _final_body(s0_ref, s10_ref, s11_ref, a0_ref, a1_ref, o_ref):
    s0 = s0_ref[...]
    s1 = s10_ref[...] + s11_ref[...]
    a = a0_ref[...] + a1_ref[...]
    tot = jnp.sum(s0) + jnp.sum(s1) + jnp.sum(s1 * a)
    o_ref[...] = jnp.reshape(tot, (1, 1))


def _final_reduce(s0, s1p0, s1p1, ap0, ap1):
    return pl.pallas_call(
        _final_body,
        out_shape=jax.ShapeDtypeStruct((1, 1), jnp.float32),
    )(s0, s1p0, s1p1, ap0, ap1)


def kernel(adj, weight_vector, id_embedding):
    w = weight_vector[:, 0]
    s0 = _compute_s0(id_embedding)
    s1p0, s1p1, ap0, ap1 = _edge_pass(adj, w, s0)
    out = _final_reduce(s0, s1p0, s1p1, ap0, ap1)
    return out[0, 0]


# speedup vs baseline: 58.4288x; 1.2317x over previous
import jax
import jax.numpy as jnp
from jax import lax
from jax.experimental import pallas as pl
from jax.experimental.pallas import tpu as pltpu
from jax.experimental.pallas import tpu_sc as plsc

N_NODES = 50000
N_EDGES = 800000
NBLK = N_EDGES // 128
BLK_LO = NBLK // 32
BLK_HI = BLK_LO + 1
CHUNK = 3200
NFULL = 7
TAIL_HI = BLK_HI * 128 - NFULL * CHUNK
TAIL_LO = BLK_LO * 128 - NFULL * CHUNK
N_ACC = 50048
ZSLICE = N_ACC // 16
ROW_BLOCK = 1024
L = 16


def _s0_body(x_ref, o_ref):
    x = x_ref[...]
    ones = jnp.ones((1, x.shape[1]), jnp.float32)
    dn = (((1,), (1,)), ((), ()))
    rs = jax.lax.dot_general(ones, x, dn, preferred_element_type=jnp.float32)
    sq = jax.lax.dot_general(ones, x * x, dn, preferred_element_type=jnp.float32)
    o_ref[...] = rs[0] / jnp.maximum(jnp.sqrt(sq[0]), 1e-12)


def _compute_s0(emb):
    return pl.pallas_call(
        _s0_body,
        out_shape=jax.ShapeDtypeStruct((N_NODES,), jnp.float32),
    )(emb)


def _edge_body(adj_hbm, w_hbm, s0_hbm, s1p0_hbm, s1p1_hbm, ap0_hbm, ap1_hbm,
               s0_v, ed_v, src_v, dst_v, w_v, msg_v, s1_sh, a_sh):
    core = lax.axis_index("c")
    sid = lax.axis_index("s")
    wid = sid * 2 + core

    def zbody(i, _):
        msg_v[pl.ds(i * L, L)] = jnp.zeros((L,), jnp.float32)
        return 0
    lax.fori_loop(0, CHUNK // L, zbody, 0)

    zs = pl.ds(sid * ZSLICE, ZSLICE)
    pltpu.sync_copy(msg_v.at[pl.ds(0, ZSLICE)], s1_sh.at[zs])
    pltpu.sync_copy(msg_v.at[pl.ds(0, ZSLICE)], a_sh.at[zs])

    pltpu.sync_copy(s0_hbm, s0_v)
    plsc.subcore_barrier()

    base = (wid * BLK_LO + jnp.minimum(wid, 10)) * 128

    def do_chunk(off, nedge):
        pltpu.sync_copy(adj_hbm.at[:, pl.ds(off, nedge)], ed_v.at[:, pl.ds(0, nedge)])
        pltpu.sync_copy(w_hbm.at[pl.ds(off, nedge)], w_v.at[pl.ds(0, nedge)])
        def gbody(i, _):
            o = i * L
            sv = ed_v[0, pl.ds(o, L)]
            dv = ed_v[1, pl.ds(o, L)]
            src_v[pl.ds(o, L)] = sv
            dst_v[pl.ds(o, L)] = dv
            vals = plsc.load_gather(s0_v, [sv])
            msg_v[pl.ds(o, L)] = w_v[pl.ds(o, L)] * vals
            return 0
        lax.fori_loop(0, nedge // L, gbody, 0)

        pltpu.sync_copy(msg_v.at[pl.ds(0, nedge)], s1_sh.at[dst_v.at[pl.ds(0, nedge)]], add=True)
        pltpu.sync_copy(w_v.at[pl.ds(0, nedge)], a_sh.at[src_v.at[pl.ds(0, nedge)]], add=True)

    for c in range(NFULL):
        do_chunk(base + c * CHUNK, CHUNK)

    tail_off = base + NFULL * CHUNK

    @pl.when(wid < 10)
    def _tail_hi():
        do_chunk(tail_off, TAIL_HI)

    @pl.when(wid >= 10)
    def _tail_lo():
        do_chunk(tail_off, TAIL_LO)

    plsc.subcore_barrier()

    pltpu.sync_copy(s1_sh.at[zs], msg_v.at[pl.ds(0, ZSLICE)])
    pltpu.sync_copy(a_sh.at[zs], w_v.at[pl.ds(0, ZSLICE)])

    @pl.when(core == 0)
    def _pub0():
        pltpu.sync_copy(msg_v.at[pl.ds(0, ZSLICE)], s1p0_hbm.at[zs])
        pltpu.sync_copy(w_v.at[pl.ds(0, ZSLICE)], ap0_hbm.at[zs])

    @pl.when(core == 1)
    def _pub1():
        pltpu.sync_copy(msg_v.at[pl.ds(0, ZSLICE)], s1p1_hbm.at[zs])
        pltpu.sync_copy(w_v.at[pl.ds(0, ZSLICE)], ap1_hbm.at[zs])


def _edge_pass(adj, w, s0):
    mesh = plsc.VectorSubcoreMesh(core_axis_name="c", subcore_axis_name="s")
    f = pl.kernel(
        _edge_body,
        out_type=(
            jax.ShapeDtypeStruct((N_ACC,), jnp.float32),
            jax.ShapeDtypeStruct((N_ACC,), jnp.float32),
            jax.ShapeDtypeStruct((N_ACC,), jnp.float32),
            jax.ShapeDtypeStruct((N_ACC,), jnp.float32),
        ),
        mesh=mesh,
        compiler_params=pltpu.CompilerParams(needs_layout_passes=False),
        scratch_types=[
            pltpu.VMEM((N_NODES,), jnp.float32),
            pltpu.VMEM((2, CHUNK), jnp.int32),
            pltpu.VMEM((CHUNK,), jnp.int32),
            pltpu.VMEM((CHUNK,), jnp.int32),
            pltpu.VMEM((CHUNK,), jnp.float32),
            pltpu.VMEM((CHUNK,), jnp.float32),
            pltpu.VMEM_SHARED((N_ACC,), jnp.float32),
            pltpu.VMEM_SHARED((N_ACC,), jnp.float32),
        ],
    )
    return f(adj, w, s0)


def _final_body(s0_ref, s10_ref, s11_ref, a0_ref, a1_ref, o_ref):
    s0 = s0_ref[...]
    s1 = s10_ref[...] + s11_ref[...]
    a = a0_ref[...] + a1_ref[...]
    tot = jnp.sum(s0) + jnp.sum(s1) + jnp.sum(s1 * a)
    o_ref[...] = jnp.reshape(tot, (1, 1))


def _final_reduce(s0, s1p0, s1p1, ap0, ap1):
    return pl.pallas_call(
        _final_body,
        out_shape=jax.ShapeDtypeStruct((1, 1), jnp.float32),
    )(s0, s1p0, s1p1, ap0, ap1)


def kernel(adj, weight_vector, id_embedding):
    w = weight_vector[:, 0]
    s0 = _compute_s0(id_embedding)
    s1p0, s1p1, ap0, ap1 = _edge_pass(adj, w, s0)
    out = _final_reduce(s0, s1p0, s1p1, ap0, ap1)
    return out[0, 0]
